# baseline (speedup 1.0000x reference)
import functools
import math

import jax
import jax.numpy as jnp
from jax import lax
from jax.experimental import pallas as pl
from jax.experimental.pallas import tpu as pltpu
from jax.experimental.pallas import tpu_sc as plsc

VOCAB = 100000
D = 768
NUM_LANG = 4
B = 4
S = 4096
SCALE = math.sqrt(D)

NC = 2
NS = 16
NW = NC * NS
S_PER_W = S // NW
CHUNK = 32
N_SCHUNK = S_PER_W // CHUNK
DV = D // 16


def _pos_table():
    pos = jnp.arange(0, S, dtype=jnp.float32)[:, None]
    div = jnp.exp(jnp.arange(0, D, 2, dtype=jnp.float32) * (-math.log(10000.0) / D))
    pe = jnp.zeros((S, D), dtype=jnp.float32)
    pe = pe.at[:, 0::2].set(jnp.sin(pos * div))
    pe = pe.at[:, 1::2].set(jnp.cos(pos * div))
    return pe


def _proj_body(wl_ref, wp_ref, o_ref):
    o_ref[...] = lax.dot_general(
        wl_ref[...], wp_ref[...], (((1,), (1,)), ((), ())),
        preferred_element_type=jnp.float32)


_lang_proj = pl.pallas_call(
    _proj_body,
    out_shape=jax.ShapeDtypeStruct((NUM_LANG, D), jnp.float32),
)


@functools.partial(
    pl.kernel,
    out_type=jax.ShapeDtypeStruct((B * S, D), jnp.float32),
    mesh=plsc.VectorSubcoreMesh(core_axis_name="c", subcore_axis_name="s"),
    scratch_types=[
        pltpu.VMEM((CHUNK,), jnp.int32),
        pltpu.VMEM((CHUNK,), jnp.int32),
        pltpu.VMEM((CHUNK, D), jnp.float32),
        pltpu.VMEM((CHUNK, D), jnp.float32),
        pltpu.VMEM((CHUNK, D), jnp.float32),
        pltpu.SemaphoreType.DMA,
        pltpu.SemaphoreType.DMA,
    ],
)
def _sc_embed(wtok_hbm, tokid_hbm, langid_hbm, ltab_hbm, pe_hbm, out_hbm,
              idx_v, lid_v, rows_v, lang_v, pe_v, sem, sem2):
    wid = lax.axis_index("s") * NC + lax.axis_index("c")
    s_base = wid * S_PER_W
    for sc in range(N_SCHUNK):
        s0 = s_base + sc * CHUNK
        pltpu.sync_copy(pe_hbm.at[pl.ds(s0, CHUNK)], pe_v)
        for b in range(B):
            flat0 = b * S + s0
            pltpu.sync_copy(tokid_hbm.at[pl.ds(flat0, CHUNK)], idx_v)
            pltpu.sync_copy(langid_hbm.at[pl.ds(flat0, CHUNK)], lid_v)
            tok_cp = pltpu.async_copy(wtok_hbm.at[idx_v], rows_v, sem)
            lang_cp = pltpu.async_copy(ltab_hbm.at[lid_v], lang_v, sem2)
            tok_cp.wait()
            lang_cp.wait()

            def row_body(i, _):
                for j in range(DV):
                    sl = pl.ds(j * 16, 16)
                    rows_v[i, sl] = (rows_v[i, sl] * SCALE
                                     + lang_v[i, sl] + pe_v[i, sl])
                return 0

            lax.fori_loop(0, CHUNK, row_body, 0)
            pltpu.sync_copy(rows_v, out_hbm.at[pl.ds(flat0, CHUNK)])


def kernel(token_ids, lang_ids, W_tok, W_lang, W_proj):
    tok_flat = token_ids.reshape(-1).astype(jnp.int32)
    lang_flat = lang_ids.reshape(-1).astype(jnp.int32)
    ltab = _lang_proj(W_lang, W_proj)
    pe = _pos_table()
    out = _sc_embed(W_tok, tok_flat, lang_flat, ltab, pe)
    return out.reshape(B, S, D)


# speedup vs baseline: 2.4797x; 2.4797x over previous
import functools
import math

import jax
import jax.numpy as jnp
from jax import lax
from jax.experimental import pallas as pl
from jax.experimental.pallas import tpu as pltpu
from jax.experimental.pallas import tpu_sc as plsc

VOCAB = 100000
D = 768
NUM_LANG = 4
B = 4
S = 4096
N = B * S
SCALE = math.sqrt(D)

NC = 2
NS = 16
NW = NC * NS
R_PER_W = N // NW
CHUNK = 32
NBUF = 4
NCHUNK = R_PER_W // CHUNK

BLK = 512
NBLK = S // BLK


def _pos_table():
    pos = jnp.arange(0, S, dtype=jnp.float32)[:, None]
    div = jnp.exp(jnp.arange(0, D, 2, dtype=jnp.float32) * (-math.log(10000.0) / D))
    pe = jnp.zeros((S, D), dtype=jnp.float32)
    pe = pe.at[:, 0::2].set(jnp.sin(pos * div))
    pe = pe.at[:, 1::2].set(jnp.cos(pos * div))
    return pe


def _proj_body(wl_ref, wp_ref, o_ref):
    o_ref[...] = lax.dot_general(
        wl_ref[...], wp_ref[...], (((1,), (1,)), ((), ())),
        preferred_element_type=jnp.float32)


_lang_proj = pl.pallas_call(
    _proj_body,
    out_shape=jax.ShapeDtypeStruct((NUM_LANG, D), jnp.float32),
)


@functools.partial(
    pl.kernel,
    out_type=jax.ShapeDtypeStruct((N, D), jnp.float32),
    mesh=plsc.VectorSubcoreMesh(core_axis_name="c", subcore_axis_name="s"),
    scratch_types=(
        [pltpu.VMEM((R_PER_W,), jnp.int32)]
        + [pltpu.VMEM((CHUNK, D), jnp.float32) for _ in range(NBUF)]
        + [pltpu.SemaphoreType.DMA for _ in range(2 * NBUF)]
    ),
)
def _sc_gather(wtok_hbm, tokid_hbm, g_hbm, idx_v, *bufs_and_sems):
    bufs = bufs_and_sems[:NBUF]
    gsems = bufs_and_sems[NBUF:2 * NBUF]
    ssems = bufs_and_sems[2 * NBUF:]
    wid = lax.axis_index("s") * NC + lax.axis_index("c")
    base = wid * R_PER_W
    pltpu.sync_copy(tokid_hbm.at[pl.ds(base, R_PER_W)], idx_v)
    g_cp = [None] * NCHUNK
    st_cp = [None] * NCHUNK
    for c in range(NCHUNK + 1):
        if c < NCHUNK:
            k = c % NBUF
            if c >= NBUF:
                st_cp[c - NBUF].wait()
            g_cp[c] = pltpu.async_copy(
                wtok_hbm.at[idx_v.at[pl.ds(c * CHUNK, CHUNK)]], bufs[k],
                gsems[k])
        if c >= 1:
            cc = c - 1
            kk = cc % NBUF
            g_cp[cc].wait()
            st_cp[cc] = pltpu.async_copy(
                bufs[kk], g_hbm.at[pl.ds(base + cc * CHUNK, CHUNK)],
                ssems[kk])
    for c in range(NCHUNK - NBUF, NCHUNK):
        st_cp[c].wait()


def _combine_body(lid_ref, ltab_ref, g_ref, pe_ref, o_ref):
    ids_row = lid_ref[0]
    oh = (lax.broadcasted_iota(jnp.int32, (NUM_LANG, BLK), 0)
          == jnp.broadcast_to(ids_row, (NUM_LANG, BLK))).astype(jnp.float32)
    lang = lax.dot_general(oh, ltab_ref[...], (((0,), (0,)), ((), ())),
                           preferred_element_type=jnp.float32)
    o_ref[...] = g_ref[...] * SCALE + lang + pe_ref[...]


_combine = pl.pallas_call(
    _combine_body,
    grid=(NBLK, B),
    in_specs=[
        pl.BlockSpec((1, 1, BLK), lambda i, b: (b * NBLK + i, 0, 0)),
        pl.BlockSpec((NUM_LANG, D), lambda i, b: (0, 0)),
        pl.BlockSpec((BLK, D), lambda i, b: (b * NBLK + i, 0)),
        pl.BlockSpec((BLK, D), lambda i, b: (i, 0)),
    ],
    out_specs=pl.BlockSpec((BLK, D), lambda i, b: (b * NBLK + i, 0)),
    out_shape=jax.ShapeDtypeStruct((N, D), jnp.float32),
)


def kernel(token_ids, lang_ids, W_tok, W_lang, W_proj):
    tok_flat = token_ids.reshape(-1).astype(jnp.int32)
    lang_r = lang_ids.reshape(-1).astype(jnp.int32).reshape(B * NBLK, 1, BLK)
    ltab = _lang_proj(W_lang, W_proj)
    pe = _pos_table()
    g = _sc_gather(W_tok, tok_flat)
    out = _combine(lang_r, ltab, g, pe)
    return out.reshape(B, S, D)


# speedup vs baseline: 2.5915x; 1.0451x over previous
import functools
import math

import jax
import jax.numpy as jnp
from jax import lax
from jax.experimental import pallas as pl
from jax.experimental.pallas import tpu as pltpu
from jax.experimental.pallas import tpu_sc as plsc

VOCAB = 100000
D = 768
NUM_LANG = 4
B = 4
S = 4096
N = B * S
SCALE = math.sqrt(D)

NC = 2
NS = 16
NW = NC * NS
R_PER_W = N // NW
CHUNK = 32
NBUF = 4
NCHUNK = R_PER_W // CHUNK

BLK = 1024
NBLK = S // BLK


def _pos_table():
    pos = jnp.arange(0, S, dtype=jnp.float32)[:, None]
    div = jnp.exp(jnp.arange(0, D, 2, dtype=jnp.float32) * (-math.log(10000.0) / D))
    pe = jnp.zeros((S, D), dtype=jnp.float32)
    pe = pe.at[:, 0::2].set(jnp.sin(pos * div))
    pe = pe.at[:, 1::2].set(jnp.cos(pos * div))
    return pe


def _proj_body(wl_ref, wp_ref, o_ref):
    o_ref[...] = lax.dot_general(
        wl_ref[...], wp_ref[...], (((1,), (1,)), ((), ())),
        preferred_element_type=jnp.float32)


_lang_proj = pl.pallas_call(
    _proj_body,
    out_shape=jax.ShapeDtypeStruct((NUM_LANG, D), jnp.float32),
)


@functools.partial(
    pl.kernel,
    out_type=jax.ShapeDtypeStruct((N, D), jnp.float32),
    mesh=plsc.VectorSubcoreMesh(core_axis_name="c", subcore_axis_name="s"),
    scratch_types=(
        [pltpu.VMEM((R_PER_W,), jnp.int32)]
        + [pltpu.VMEM((CHUNK, D), jnp.float32) for _ in range(NBUF)]
        + [pltpu.SemaphoreType.DMA for _ in range(2 * NBUF)]
    ),
)
def _sc_gather(wtok_hbm, tokid_hbm, g_hbm, idx_v, *bufs_and_sems):
    bufs = bufs_and_sems[:NBUF]
    gsems = bufs_and_sems[NBUF:2 * NBUF]
    ssems = bufs_and_sems[2 * NBUF:]
    wid = lax.axis_index("s") * NC + lax.axis_index("c")
    base = wid * R_PER_W
    pltpu.sync_copy(tokid_hbm.at[pl.ds(base, R_PER_W)], idx_v)
    g_cp = [None] * NCHUNK
    st_cp = [None] * NCHUNK
    for c in range(NCHUNK + 1):
        if c < NCHUNK:
            k = c % NBUF
            if c >= NBUF:
                st_cp[c - NBUF].wait()
            g_cp[c] = pltpu.async_copy(
                wtok_hbm.at[idx_v.at[pl.ds(c * CHUNK, CHUNK)]], bufs[k],
                gsems[k])
        if c >= 1:
            cc = c - 1
            kk = cc % NBUF
            g_cp[cc].wait()
            st_cp[cc] = pltpu.async_copy(
                bufs[kk], g_hbm.at[pl.ds(base + cc * CHUNK, CHUNK)],
                ssems[kk])
    for c in range(NCHUNK - NBUF, NCHUNK):
        st_cp[c].wait()


def _combine_body(lid_ref, ltab_ref, g_ref, pe_ref, o_ref):
    ids_row = lid_ref[0]
    oh = (lax.broadcasted_iota(jnp.int32, (NUM_LANG, BLK), 0)
          == jnp.broadcast_to(ids_row, (NUM_LANG, BLK))).astype(jnp.float32)
    lang = lax.dot_general(oh, ltab_ref[...], (((0,), (0,)), ((), ())),
                           preferred_element_type=jnp.float32)
    o_ref[...] = g_ref[...] * SCALE + lang + pe_ref[...]


_combine = pl.pallas_call(
    _combine_body,
    grid=(NBLK, B),
    in_specs=[
        pl.BlockSpec((1, 1, BLK), lambda i, b: (b * NBLK + i, 0, 0)),
        pl.BlockSpec((NUM_LANG, D), lambda i, b: (0, 0)),
        pl.BlockSpec((BLK, D), lambda i, b: (b * NBLK + i, 0)),
        pl.BlockSpec((BLK, D), lambda i, b: (i, 0)),
    ],
    out_specs=pl.BlockSpec((BLK, D), lambda i, b: (b * NBLK + i, 0)),
    out_shape=jax.ShapeDtypeStruct((N, D), jnp.float32),
)


def kernel(token_ids, lang_ids, W_tok, W_lang, W_proj):
    tok_flat = token_ids.reshape(-1).astype(jnp.int32)
    lang_r = lang_ids.reshape(-1).astype(jnp.int32).reshape(B * NBLK, 1, BLK)
    ltab = _lang_proj(W_lang, W_proj)
    pe = _pos_table()
    g = _sc_gather(W_tok, tok_flat)
    out = _combine(lang_r, ltab, g, pe)
    return out.reshape(B, S, D)
